# Initial kernel scaffold; baseline (speedup 1.0000x reference)
import jax
import jax.numpy as jnp
from jax.experimental import pallas as pl


def kernel(x, edge_index, batch, params, Wl, bl):
    raise NotImplementedError("write your pallas kernel here")


# speedup vs baseline: 2.8657x; 2.8657x over previous
import functools

import jax
import jax.numpy as jnp
from jax import lax
from jax.experimental import pallas as pl
from jax.experimental.pallas import tpu as pltpu
from jax.experimental.pallas import tpu_sc as plsc

_NC = 2
_NS = 16
_NW = _NC * _NS
_CH = 128


def _sc_agg_body(nchunk, npad, d,
                 h_hbm, src_hbm, dst_hbm, out_hbm,
                 src_v, dst_v, rows0, rows1, acc, sem0, sem1):
    c = lax.axis_index("c")
    s = lax.axis_index("s")
    wid = s * _NC + c

    rows_per = npad // _NS
    base = s * rows_per

    for j in range(d // 16):
        def zstep(i, _, j=j):
            rows0[i, pl.ds(j * 16, 16)] = jnp.zeros((16,), jnp.float32)
            return _
        lax.fori_loop(0, _CH, zstep, 0)
    for t in range(rows_per // _CH):
        pltpu.sync_copy(rows0, acc.at[pl.ds(base + t * _CH, _CH)])
    plsc.subcore_barrier()

    def fire(j, buf, sem):
        pltpu.async_copy(h_hbm.at[src_v.at[j]], buf, sem)

    def drain(j, buf, sem):
        pltpu.make_async_copy(h_hbm.at[src_v.at[j]], buf, sem).wait()

    half = nchunk // 2
    for hh in range(2):
        pltpu.sync_copy(src_hbm.at[wid, pl.ds(hh * half, half)], src_v)
        pltpu.sync_copy(dst_hbm.at[wid, pl.ds(hh * half, half)], dst_v)

        fire(0, rows0, sem0)

        def step(i, carry):
            a = 2 * i
            b = 2 * i + 1
            fire(b, rows1, sem1)
            drain(a, rows0, sem0)
            pltpu.sync_copy(rows0, acc.at[dst_v.at[a]], add=True)

            @pl.when(b + 1 < half)
            def _():
                fire(b + 1, rows0, sem0)

            drain(b, rows1, sem1)
            pltpu.sync_copy(rows1, acc.at[dst_v.at[b]], add=True)
            return carry

        lax.fori_loop(0, half // 2, step, 0)

    plsc.subcore_barrier()
    pltpu.sync_copy(acc.at[pl.ds(base, rows_per)],
                    out_hbm.at[c, pl.ds(base, rows_per)])


def _mlp_bn(z, w1, b1, w2, b2, g, be):
    hi = jax.lax.Precision.HIGHEST
    z = jnp.maximum(jnp.dot(z, w1[...], precision=hi,
                            preferred_element_type=jnp.float32) + b1[...], 0.0)
    z = jnp.dot(z, w2[...], precision=hi,
                preferred_element_type=jnp.float32) + b2[...]
    mu = jnp.mean(z, axis=0, keepdims=True)
    zc = z - mu
    var = jnp.mean(zc * zc, axis=0, keepdims=True)
    z = zc * jax.lax.rsqrt(var + 1e-5) * g[...] + be[...]
    return jnp.maximum(z, 0.0)


def _tc_layer_body(n, h_ref, parts_ref, w1, b1, w2, b2, g, be, out_ref):
    z = h_ref[...] + parts_ref[0, :n, :] + parts_ref[1, :n, :]
    out_ref[...] = _mlp_bn(z, w1, b1, w2, b2, g, be)


def _tc_final_body(n, gseg, h_ref, parts_ref, w1, b1, w2, b2, g, be,
                   batch_ref, wl, bl, out_ref):
    z = h_ref[...] + parts_ref[0, :n, :] + parts_ref[1, :n, :]
    h3 = _mlp_bn(z, w1, b1, w2, b2, g, be)
    ids = batch_ref[...]
    iot = lax.broadcasted_iota(jnp.int32, (gseg, n), 0)
    sel = jnp.where(iot == ids, 1.0, 0.0)
    cnt = jnp.sum(sel, axis=1, keepdims=True)
    hi = jax.lax.Precision.HIGHEST
    sums = jnp.dot(sel, h3, precision=hi,
                   preferred_element_type=jnp.float32)
    pooled = sums / jnp.maximum(cnt, 1.0)
    out_ref[...] = jnp.dot(pooled, wl[...], precision=hi,
                           preferred_element_type=jnp.float32) + bl[...]


def kernel(x, edge_index, batch, params, Wl, bl):
    n, d = x.shape
    e = edge_index.shape[1]
    gseg = 64

    nchunk = -(-e // (_NW * _CH))
    nchunk = -(-nchunk // 4) * 4
    epad = _NW * _CH * nchunk - e
    rows_per = _CH * (-(-(n + 1) // (_NS * _CH)))
    npad = _NS * rows_per

    src = jnp.concatenate([edge_index[0], jnp.zeros((epad,), jnp.int32)])
    dst = jnp.concatenate([edge_index[1], jnp.full((epad,), n, jnp.int32)])
    srcp = src.reshape(_NW, nchunk, _CH)
    dstp = dst.reshape(_NW, nchunk, _CH)

    sc_agg = pl.kernel(
        functools.partial(_sc_agg_body, nchunk, npad, d),
        out_type=jax.ShapeDtypeStruct((_NC, npad, d), jnp.float32),
        mesh=plsc.VectorSubcoreMesh(core_axis_name="c", subcore_axis_name="s"),
        scratch_types=[
            pltpu.VMEM((nchunk // 2, _CH), jnp.int32),
            pltpu.VMEM((nchunk // 2, _CH), jnp.int32),
            pltpu.VMEM((_CH, d), jnp.float32),
            pltpu.VMEM((_CH, d), jnp.float32),
            pltpu.VMEM_SHARED((npad, d), jnp.float32),
            pltpu.SemaphoreType.DMA,
            pltpu.SemaphoreType.DMA,
        ],
    )

    h = x
    for i, (W1, b1, W2, b2, gamma, beta) in enumerate(params):
        parts = sc_agg(h, srcp, dstp)
        wargs = (W1, b1.reshape(1, -1), W2, b2.reshape(1, -1),
                 gamma.reshape(1, -1), beta.reshape(1, -1))
        if i + 1 < len(params):
            h = pl.pallas_call(
                functools.partial(_tc_layer_body, n),
                out_shape=jax.ShapeDtypeStruct((n, W2.shape[1]), jnp.float32),
            )(h, parts, *wargs)
        else:
            out = pl.pallas_call(
                functools.partial(_tc_final_body, n, gseg),
                out_shape=jax.ShapeDtypeStruct((gseg, Wl.shape[1]),
                                               jnp.float32),
            )(h, parts, *wargs, batch.reshape(1, n), Wl, bl.reshape(1, -1))
    return out


# speedup vs baseline: 3.1619x; 1.1034x over previous
import functools

import jax
import jax.numpy as jnp
from jax import lax
from jax.experimental import pallas as pl
from jax.experimental.pallas import tpu as pltpu
from jax.experimental.pallas import tpu_sc as plsc

_NC = 2
_NS = 16
_NW = _NC * _NS
_CH = 128


def _sc_agg_body(nchunk, npad, d,
                 h_hbm, src_hbm, dst_hbm, out_hbm,
                 src_v, dst_v, rows0, rows1, acc, sem0, sem1):
    c = lax.axis_index("c")
    s = lax.axis_index("s")
    wid = s * _NC + c

    rows_per = npad // _NS
    base = s * rows_per

    for j in range(d // 16):
        def zstep(i, _, j=j):
            rows0[i, pl.ds(j * 16, 16)] = jnp.zeros((16,), jnp.float32)
            return _
        lax.fori_loop(0, _CH, zstep, 0)
    for t in range(rows_per // _CH):
        pltpu.sync_copy(rows0, acc.at[pl.ds(base + t * _CH, _CH)])
    plsc.subcore_barrier()

    def fire(j, buf, sem):
        pltpu.async_copy(h_hbm.at[src_v.at[j]], buf, sem)

    def drain(j, buf, sem):
        pltpu.make_async_copy(h_hbm.at[src_v.at[j]], buf, sem).wait()

    half = nchunk // 2
    for hh in range(2):
        pltpu.sync_copy(src_hbm.at[wid, pl.ds(hh * half, half)], src_v)
        pltpu.sync_copy(dst_hbm.at[wid, pl.ds(hh * half, half)], dst_v)

        fire(0, rows0, sem0)

        def step(i, carry):
            a = 2 * i
            b = 2 * i + 1
            fire(b, rows1, sem1)
            drain(a, rows0, sem0)
            pltpu.sync_copy(rows0, acc.at[dst_v.at[a]], add=True)

            @pl.when(b + 1 < half)
            def _():
                fire(b + 1, rows0, sem0)

            drain(b, rows1, sem1)
            pltpu.sync_copy(rows1, acc.at[dst_v.at[b]], add=True)
            return carry

        lax.fori_loop(0, half // 2, step, 0)

    plsc.subcore_barrier()
    pltpu.sync_copy(acc.at[pl.ds(base, rows_per)],
                    out_hbm.at[c, pl.ds(base, rows_per)])


def _mlp_bn(z, w1, b1, w2, b2, g, be):
    hi = jax.lax.Precision.HIGHEST
    z = jnp.maximum(jnp.dot(z, w1[...], precision=hi,
                            preferred_element_type=jnp.float32) + b1[...], 0.0)
    z = jnp.dot(z, w2[...], precision=hi,
                preferred_element_type=jnp.float32) + b2[...]
    mu = jnp.mean(z, axis=0, keepdims=True)
    zc = z - mu
    var = jnp.mean(zc * zc, axis=0, keepdims=True)
    z = zc * jax.lax.rsqrt(var + 1e-5) * g[...] + be[...]
    return jnp.maximum(z, 0.0)


def _tc_layer_body(n, h_ref, parts_ref, w1, b1, w2, b2, g, be, out_ref):
    z = h_ref[...] + parts_ref[0, :n, :] + parts_ref[1, :n, :]
    out_ref[...] = _mlp_bn(z, w1, b1, w2, b2, g, be)


def _tc_final_body(n, gseg, h_ref, parts_ref, w1, b1, w2, b2, g, be,
                   batch_ref, wl, bl, out_ref):
    z = h_ref[...] + parts_ref[0, :n, :] + parts_ref[1, :n, :]
    h3 = _mlp_bn(z, w1, b1, w2, b2, g, be)
    ids = batch_ref[...]
    iot = lax.broadcasted_iota(jnp.int32, (gseg, n), 0)
    sel = jnp.where(iot == ids, 1.0, 0.0)
    cnt = jnp.sum(sel, axis=1, keepdims=True)
    hi = jax.lax.Precision.HIGHEST
    sums = jnp.dot(sel, h3, precision=hi,
                   preferred_element_type=jnp.float32)
    pooled = sums / jnp.maximum(cnt, 1.0)
    out_ref[...] = jnp.dot(pooled, wl[...], precision=hi,
                           preferred_element_type=jnp.float32) + bl[...]


def kernel(x, edge_index, batch, params, Wl, bl):
    n, d = x.shape
    e = edge_index.shape[1]
    gseg = 64

    nchunk = -(-e // (_NW * _CH))
    nchunk = -(-nchunk // 4) * 4
    epad = _NW * _CH * nchunk - e
    rows_per = _CH * (-(-(n + 1) // (_NS * _CH)))
    npad = _NS * rows_per

    pad_dst = n + jax.lax.rem(jnp.arange(epad, dtype=jnp.int32),
                              jnp.int32(npad - n))
    src = jnp.concatenate([edge_index[0], jnp.zeros((epad,), jnp.int32)])
    dst = jnp.concatenate([edge_index[1], pad_dst])
    srcp = src.reshape(_NW, nchunk, _CH)
    dstp = dst.reshape(_NW, nchunk, _CH)

    sc_agg = pl.kernel(
        functools.partial(_sc_agg_body, nchunk, npad, d),
        out_type=jax.ShapeDtypeStruct((_NC, npad, d), jnp.float32),
        mesh=plsc.VectorSubcoreMesh(core_axis_name="c", subcore_axis_name="s"),
        scratch_types=[
            pltpu.VMEM((nchunk // 2, _CH), jnp.int32),
            pltpu.VMEM((nchunk // 2, _CH), jnp.int32),
            pltpu.VMEM((_CH, d), jnp.float32),
            pltpu.VMEM((_CH, d), jnp.float32),
            pltpu.VMEM_SHARED((npad, d), jnp.float32),
            pltpu.SemaphoreType.DMA,
            pltpu.SemaphoreType.DMA,
        ],
    )

    h = x
    for i, (W1, b1, W2, b2, gamma, beta) in enumerate(params):
        parts = sc_agg(h, srcp, dstp)
        wargs = (W1, b1.reshape(1, -1), W2, b2.reshape(1, -1),
                 gamma.reshape(1, -1), beta.reshape(1, -1))
        if i + 1 < len(params):
            h = pl.pallas_call(
                functools.partial(_tc_layer_body, n),
                out_shape=jax.ShapeDtypeStruct((n, W2.shape[1]), jnp.float32),
            )(h, parts, *wargs)
        else:
            out = pl.pallas_call(
                functools.partial(_tc_final_body, n, gseg),
                out_shape=jax.ShapeDtypeStruct((gseg, Wl.shape[1]),
                                               jnp.float32),
            )(h, parts, *wargs, batch.reshape(1, n), Wl, bl.reshape(1, -1))
    return out
